# Initial kernel scaffold; baseline (speedup 1.0000x reference)
import jax
import jax.numpy as jnp
from jax.experimental import pallas as pl


def kernel(sentences, word_vectors, W, b):
    raise NotImplementedError("write your pallas kernel here")


# speedup vs baseline: 2.7890x; 2.7890x over previous
import functools

import jax
import jax.numpy as jnp
from jax import lax
from jax.experimental import pallas as pl
from jax.experimental.pallas import tpu as pltpu
from jax.experimental.pallas import tpu_sc as plsc


def _rowsum_body(wv_ref, out_ref, *, inv_l, blk_rows, d):
    x = wv_ref[...]
    ones = jnp.full((1, d), inv_l, dtype=jnp.float32)
    parts = []
    for c in range(blk_rows // 128):
        xc = x[c * 128:(c + 1) * 128, :]
        parts.append(lax.dot_general(
            ones, xc, (((1,), (1,)), ((), ())),
            preferred_element_type=jnp.float32))
    out_ref[...] = jnp.concatenate(parts, axis=0)


def _rowsum(word_vectors, L):
    voc, d = word_vectors.shape
    blk_rows = 4096
    grid = pl.cdiv(voc, blk_rows)
    out_rows = 8192
    assert grid * (blk_rows // 128) <= out_rows
    return pl.pallas_call(
        functools.partial(_rowsum_body, inv_l=1.0 / float(L),
                          blk_rows=blk_rows, d=d),
        grid=(grid,),
        in_specs=[pl.BlockSpec((blk_rows, d), lambda i: (i, 0))],
        out_specs=pl.BlockSpec((blk_rows // 128, 128), lambda i: (i, 0)),
        out_shape=jax.ShapeDtypeStruct((out_rows, 128), jnp.float32),
    )(word_vectors)


def _make_sc_gather(n_idx):
    info = plsc.get_sparse_core_info()
    nc, ns = info.num_cores, info.num_subcores
    nw = nc * ns
    assert n_idx % nw == 0
    per_w = n_idx // nw
    mesh = plsc.VectorSubcoreMesh(core_axis_name="c", subcore_axis_name="s")

    @functools.partial(
        pl.kernel,
        out_type=jax.ShapeDtypeStruct((n_idx,), jnp.float32),
        mesh=mesh,
        scratch_types=[
            pltpu.VMEM((per_w,), jnp.int32),
            pltpu.VMEM((per_w,), jnp.float32),
            pltpu.SemaphoreType.DMA,
        ],
    )
    def gather_k(rowsum_hbm, idx_hbm, out_hbm, idx_v, val_v, sem):
        wid = lax.axis_index("s") * nc + lax.axis_index("c")
        base = wid * per_w
        pltpu.sync_copy(idx_hbm.at[pl.ds(base, per_w)], idx_v)
        pltpu.async_copy(rowsum_hbm.at[idx_v], val_v, sem).wait()
        pltpu.sync_copy(val_v, out_hbm.at[pl.ds(base, per_w)])

    return gather_k


def _finish_body(g_ref, w_ref, b_ref, out_ref):
    s = jnp.sum(g_ref[...], axis=1)
    wsum = jnp.sum(w_ref[...], axis=1)
    out_ref[...] = s[:, None] * wsum[None, :] + b_ref[...]


def _finish(gathered, W, b):
    bsz, L = gathered.shape
    n_labels, d = W.shape
    blk_b = 1024
    return pl.pallas_call(
        _finish_body,
        grid=(bsz // blk_b,),
        in_specs=[
            pl.BlockSpec((blk_b, L), lambda i: (i, 0)),
            pl.BlockSpec((n_labels, d), lambda i: (0, 0)),
            pl.BlockSpec((1, n_labels), lambda i: (0, 0)),
        ],
        out_specs=pl.BlockSpec((blk_b, n_labels), lambda i: (i, 0)),
        out_shape=jax.ShapeDtypeStruct((bsz, n_labels), jnp.float32),
    )(gathered, W, b.reshape(1, n_labels))


def kernel(sentences, word_vectors, W, b):
    bsz, L = sentences.shape
    rowsum = _rowsum(word_vectors, L).reshape(-1)
    idx = sentences.reshape(-1).astype(jnp.int32)
    vals = _make_sc_gather(bsz * L)(rowsum, idx)
    return _finish(vals.reshape(bsz, L), W, b)


# speedup vs baseline: 3.2846x; 1.1777x over previous
import functools

import jax
import jax.numpy as jnp
from jax import lax
from jax.experimental import pallas as pl
from jax.experimental.pallas import tpu as pltpu
from jax.experimental.pallas import tpu_sc as plsc


def _rowsum_body(wv_ref, out_ref, *, inv_l, blk_rows, d):
    x = wv_ref[...]
    ones = jnp.full((1, d), inv_l, dtype=jnp.float32)
    parts = []
    for c in range(blk_rows // 128):
        xc = x[c * 128:(c + 1) * 128, :]
        parts.append(lax.dot_general(
            ones, xc, (((1,), (1,)), ((), ())),
            preferred_element_type=jnp.float32))
    out_ref[...] = jnp.concatenate(parts, axis=0)


def _rowsum(word_vectors, L):
    voc, d = word_vectors.shape
    blk_rows = 16384
    grid = pl.cdiv(voc, blk_rows)
    out_rows = 8192
    assert grid * (blk_rows // 128) <= out_rows
    return pl.pallas_call(
        functools.partial(_rowsum_body, inv_l=1.0 / float(L),
                          blk_rows=blk_rows, d=d),
        grid=(grid,),
        in_specs=[pl.BlockSpec((blk_rows, d), lambda i: (i, 0))],
        out_specs=pl.BlockSpec((blk_rows // 128, 128), lambda i: (i, 0)),
        out_shape=jax.ShapeDtypeStruct((out_rows, 128), jnp.float32),
    )(word_vectors)


def _make_sc_gather(n_idx):
    info = plsc.get_sparse_core_info()
    nc, ns = info.num_cores, info.num_subcores
    nw = nc * ns
    assert n_idx % nw == 0
    per_w = n_idx // nw
    mesh = plsc.VectorSubcoreMesh(core_axis_name="c", subcore_axis_name="s")

    @functools.partial(
        pl.kernel,
        out_type=jax.ShapeDtypeStruct((n_idx,), jnp.float32),
        mesh=mesh,
        scratch_types=[
            pltpu.VMEM((per_w,), jnp.int32),
            pltpu.VMEM((per_w,), jnp.float32),
            pltpu.SemaphoreType.DMA,
        ],
    )
    def gather_k(rowsum_hbm, idx_hbm, out_hbm, idx_v, val_v, sem):
        wid = lax.axis_index("s") * nc + lax.axis_index("c")
        base = wid * per_w
        pltpu.sync_copy(idx_hbm.at[pl.ds(base, per_w)], idx_v)
        pltpu.async_copy(rowsum_hbm.at[idx_v], val_v, sem).wait()
        pltpu.sync_copy(val_v, out_hbm.at[pl.ds(base, per_w)])

    return gather_k


def _finish_body(g_ref, w_ref, b_ref, out_ref):
    s = jnp.sum(g_ref[...], axis=1)
    wsum = jnp.sum(w_ref[...], axis=1)
    out_ref[...] = s[:, None] * wsum[None, :] + b_ref[...]


def _finish(gathered, W, b):
    bsz, L = gathered.shape
    n_labels, d = W.shape
    blk_b = 1024
    return pl.pallas_call(
        _finish_body,
        grid=(bsz // blk_b,),
        in_specs=[
            pl.BlockSpec((blk_b, L), lambda i: (i, 0)),
            pl.BlockSpec((n_labels, d), lambda i: (0, 0)),
            pl.BlockSpec((1, n_labels), lambda i: (0, 0)),
        ],
        out_specs=pl.BlockSpec((blk_b, n_labels), lambda i: (i, 0)),
        out_shape=jax.ShapeDtypeStruct((bsz, n_labels), jnp.float32),
    )(gathered, W, b.reshape(1, n_labels))


def kernel(sentences, word_vectors, W, b):
    bsz, L = sentences.shape
    rowsum = _rowsum(word_vectors, L).reshape(-1)
    idx = sentences.reshape(-1).astype(jnp.int32)
    vals = _make_sc_gather(bsz * L)(rowsum, idx)
    return _finish(vals.reshape(bsz, L), W, b)
